# Initial kernel scaffold; baseline (speedup 1.0000x reference)
import jax
import jax.numpy as jnp
from jax.experimental import pallas as pl


def kernel(x, edge_index, W, b):
    raise NotImplementedError("write your pallas kernel here")


# speedup vs baseline: 28.0010x; 28.0010x over previous
import jax
import jax.numpy as jnp
from jax import lax
from jax.experimental import pallas as pl
from jax.experimental.pallas import tpu as pltpu
from jax.experimental.pallas import tpu_sc as plsc

NC = 2
NS = 16
NW = NC * NS
LANES = 16
CHUNK = 128
NBUF = 4
HD = 64


def _sc_mesh():
    return plsc.VectorSubcoreMesh(
        core_axis_name="c", subcore_axis_name="s", num_cores=NC, num_subcores=NS
    )


def _deg_body(col_hbm, deg_hbm, colv, degv):
    cid = lax.axis_index("c")
    sid = lax.axis_index("s")
    gt = sid * NC + cid
    pltpu.sync_copy(col_hbm.at[gt], colv)
    n_groups = degv.shape[0] // LANES
    zero = jnp.zeros((LANES,), jnp.float32)

    def zbody(i, _):
        degv[pl.ds(i * LANES, LANES)] = zero
        return 0

    lax.fori_loop(0, n_groups, zbody, 0)
    ones = jnp.ones((LANES,), jnp.float32)
    n_vec = colv.shape[0]

    def body(i, _):
        cvec = colv[i]
        plsc.addupdate_scatter(degv, [cvec], ones)
        return 0

    lax.fori_loop(0, n_vec, body, 0)
    pltpu.sync_copy(degv, deg_hbm.at[gt])


def _make_deg_kernel(npad, n_vec):
    return pl.kernel(
        _deg_body,
        out_type=jax.ShapeDtypeStruct((NW, npad), jnp.float32),
        mesh=_sc_mesh(),
        compiler_params=pltpu.CompilerParams(needs_layout_passes=False),
        scratch_types=[
            pltpu.VMEM((n_vec, LANES), jnp.int32),
            pltpu.VMEM((npad,), jnp.float32),
        ],
    )


def _hop_body(g_hbm, row_hbm, col_hbm, acc_hbm, rowv, colv, gbuf,
              acc_sh, sem0, sem1, sem2, sem3):
    sems = (sem0, sem1, sem2, sem3)
    cid = lax.axis_index("c")
    sid = lax.axis_index("s")
    cpt = rowv.shape[0]
    npad = acc_sh.shape[0]
    rows_per_tile = npad // NS

    pltpu.sync_copy(row_hbm.at[sid], rowv)
    pltpu.sync_copy(col_hbm.at[sid], colv)

    zero = jnp.zeros((LANES,), jnp.float32)

    def zbody(i, _):
        for j in range(HD // LANES):
            gbuf[0, i, pl.ds(j * LANES, LANES)] = zero
        return 0

    lax.fori_loop(0, CHUNK, zbody, 0)
    for kk in range(rows_per_tile // CHUNK):
        pltpu.sync_copy(
            gbuf.at[0], acc_sh.at[pl.ds(sid * rows_per_tile + kk * CHUNK, CHUNK)]
        )
    plsc.subcore_barrier()

    def gather(cc, bb):
        pltpu.async_copy(g_hbm.at[cid].at[rowv.at[cc]], gbuf.at[bb], sems[bb])

    def gwait(cc, bb):
        pltpu.make_async_copy(
            g_hbm.at[cid].at[rowv.at[cc]], gbuf.at[bb], sems[bb]
        ).wait()

    gather(0, 0)
    gather(1, 1)

    def body(k, _):
        for b in range(NBUF):
            c = k * NBUF + b
            gwait(c, b)

            @pl.when(c + 2 < cpt)
            def _():
                gather(c + 2, (b + 2) % NBUF)

            pltpu.sync_copy(gbuf.at[b], acc_sh.at[colv.at[c]], add=True)
        return 0

    lax.fori_loop(0, cpt // NBUF, body, 0)
    plsc.subcore_barrier()

    for kk in range(rows_per_tile // CHUNK):
        sl = pl.ds(sid * rows_per_tile + kk * CHUNK, CHUNK)
        pltpu.sync_copy(acc_sh.at[sl], gbuf.at[0])
        pltpu.sync_copy(gbuf.at[0], acc_hbm.at[cid, sl])


def _make_hop_kernel(npad, cpt):
    return pl.kernel(
        _hop_body,
        out_type=jax.ShapeDtypeStruct((NC, npad, HD), jnp.float32),
        mesh=_sc_mesh(),
        compiler_params=pltpu.CompilerParams(
            needs_layout_passes=False, use_tc_tiling_on_sc=False
        ),
        scratch_types=[
            pltpu.VMEM((cpt, CHUNK), jnp.int32),
            pltpu.VMEM((cpt, CHUNK), jnp.int32),
            pltpu.VMEM((NBUF, CHUNK, HD), jnp.float32),
            pltpu.VMEM_SHARED((npad, HD), jnp.float32),
            pltpu.SemaphoreType.DMA,
            pltpu.SemaphoreType.DMA,
            pltpu.SemaphoreType.DMA,
            pltpu.SemaphoreType.DMA,
        ],
    )


def _scale_body(degt_ref, x_ref, g_ref, dis_ref, dinv_ref):
    d = jnp.sum(degt_ref[...], axis=1, keepdims=True) + 1.0
    dis = lax.rsqrt(d)
    dis_ref[...] = dis
    dinv_ref[...] = dis * dis
    g = x_ref[...] * dis
    g_ref[0] = g[:, :HD]
    g_ref[1] = g[:, HD:]


def _scale_call(degt, xpad, npad):
    blk = 512
    return pl.pallas_call(
        _scale_body,
        grid=(npad // blk,),
        in_specs=[
            pl.BlockSpec((blk, NW), lambda i: (i, 0)),
            pl.BlockSpec((blk, 2 * HD), lambda i: (i, 0)),
        ],
        out_specs=[
            pl.BlockSpec((NC, blk, HD), lambda i: (0, i, 0)),
            pl.BlockSpec((blk, 1), lambda i: (i, 0)),
            pl.BlockSpec((blk, 1), lambda i: (i, 0)),
        ],
        out_shape=[
            jax.ShapeDtypeStruct((NC, npad, HD), jnp.float32),
            jax.ShapeDtypeStruct((npad, 1), jnp.float32),
            jax.ShapeDtypeStruct((npad, 1), jnp.float32),
        ],
    )(degt, xpad)


def _combine_body(acc_ref, g_ref, s_ref, out_ref):
    s = s_ref[...]
    out_ref[0] = s * (acc_ref[0] + g_ref[0])
    out_ref[1] = s * (acc_ref[1] + g_ref[1])


def _combine_call(acc, g, s, npad):
    blk = 512
    return pl.pallas_call(
        _combine_body,
        grid=(npad // blk,),
        in_specs=[
            pl.BlockSpec((NC, blk, HD), lambda i: (0, i, 0)),
            pl.BlockSpec((NC, blk, HD), lambda i: (0, i, 0)),
            pl.BlockSpec((blk, 1), lambda i: (i, 0)),
        ],
        out_specs=pl.BlockSpec((NC, blk, HD), lambda i: (0, i, 0)),
        out_shape=jax.ShapeDtypeStruct((NC, npad, HD), jnp.float32),
    )(acc, g, s)


def _final_body(acc_ref, g_ref, dis_ref, wt_ref, b_ref, out_ref):
    s2 = jnp.concatenate(
        [acc_ref[0] + g_ref[0], acc_ref[1] + g_ref[1]], axis=-1
    )
    prod = jnp.dot(s2, wt_ref[...], preferred_element_type=jnp.float32)
    out_ref[...] = dis_ref[...] * prod + b_ref[...]


def _final_call(acc, g, dis, wt, bp, npad):
    blk = 512
    return pl.pallas_call(
        _final_body,
        grid=(npad // blk,),
        in_specs=[
            pl.BlockSpec((NC, blk, HD), lambda i: (0, i, 0)),
            pl.BlockSpec((NC, blk, HD), lambda i: (0, i, 0)),
            pl.BlockSpec((blk, 1), lambda i: (i, 0)),
            pl.BlockSpec((128, 128), lambda i: (0, 0)),
            pl.BlockSpec((1, 128), lambda i: (0, 0)),
        ],
        out_specs=pl.BlockSpec((blk, 128), lambda i: (i, 0)),
        out_shape=jax.ShapeDtypeStruct((npad, 128), jnp.float32),
    )(acc, g, dis, wt, bp)


def kernel(x, edge_index, W, b):
    n, d_feat = x.shape
    n_cls = W.shape[0]
    e = edge_index.shape[1]

    npad = ((n + 512 - 1) // 512) * 512
    unit = NS * CHUNK * NBUF
    epad = ((e + unit - 1) // unit) * unit
    cpt = epad // (NS * CHUNK)
    n_dummy = npad - n

    row = edge_index[0].astype(jnp.int32)
    col = edge_index[1].astype(jnp.int32)
    pad_idx = n + (jnp.arange(epad - e, dtype=jnp.int32) % n_dummy)
    row_p = jnp.concatenate([row, pad_idx])
    col_p = jnp.concatenate([col, pad_idx])
    row3 = row_p.reshape(NS, cpt, CHUNK)
    col3 = col_p.reshape(NS, cpt, CHUNK)
    col2 = col_p.reshape(NW, epad // (NW * LANES), LANES)

    xpad = jnp.zeros((npad, d_feat), jnp.float32).at[:n].set(x)
    wt = jnp.zeros((d_feat, 128), jnp.float32).at[:, :n_cls].set(W.T)
    bp = jnp.zeros((1, 128), jnp.float32).at[0, :n_cls].set(b)

    deg_parts = _make_deg_kernel(npad, epad // (NW * LANES))(col2)
    degt = deg_parts.T
    g0, dis, dinv = _scale_call(degt, xpad, npad)

    hop = _make_hop_kernel(npad, cpt)
    acc1 = hop(g0, row3, col3)
    g1 = _combine_call(acc1, g0, dinv, npad)
    acc2 = hop(g1, row3, col3)
    out_full = _final_call(acc2, g1, dis, wt, bp, npad)
    return out_full[:n, :n_cls]


# speedup vs baseline: 31.9960x; 1.1427x over previous
import jax
import jax.numpy as jnp
from jax import lax
from jax.experimental import pallas as pl
from jax.experimental.pallas import tpu as pltpu
from jax.experimental.pallas import tpu_sc as plsc

NC = 2
NS = 16
LANES = 16
CHUNK = 128
NBUF = 4
HD = 64


def _sc_mesh():
    return plsc.VectorSubcoreMesh(
        core_axis_name="c", subcore_axis_name="s", num_cores=NC, num_subcores=NS
    )


def _splat(s, dtype=jnp.float32):
    return jnp.full((LANES,), s, dtype)


def _sc_body(x_hbm, row_hbm, col_hbm, h2_hbm, g0_hbm, g1_hbm,
             rowv, colv, gbuf, degv, disv, dinvv, idr,
             acc_sh, deg_sh, sem0, sem1, sem2, sem3):
    sems = (sem0, sem1, sem2, sem3)
    cid = lax.axis_index("c")
    sid = lax.axis_index("s")
    cpt = rowv.shape[0]
    npad = acc_sh.shape[0]
    rpt = npad // NS
    nck = rpt // CHUNK
    base = sid * rpt

    pltpu.sync_copy(row_hbm.at[sid], rowv)
    pltpu.sync_copy(col_hbm.at[sid], colv)

    zero16 = jnp.zeros((LANES,), jnp.float32)

    def _zero_buf(bb):
        def _zb(i, _):
            for j in range(HD // LANES):
                gbuf[bb, i, pl.ds(j * LANES, LANES)] = zero16
            return 0

        lax.fori_loop(0, CHUNK, _zb, 0)

    _zero_buf(0)
    for kk in range(nck):
        pltpu.sync_copy(gbuf.at[0], acc_sh.at[pl.ds(base + kk * CHUNK, CHUNK)])

    def _idr(k, _):
        def _idrj(j, _):
            idr[k, pl.ds(j * LANES, LANES)] = (
                jnp.arange(LANES, dtype=jnp.int32)
                + _splat(k * CHUNK + j * LANES, jnp.int32)
            )
            return 0

        lax.fori_loop(0, CHUNK // LANES, _idrj, 0)
        return 0

    lax.fori_loop(0, idr.shape[0], _idr, 0)

    def _zd(i, _):
        degv[i] = zero16
        return 0

    lax.fori_loop(0, degv.shape[0], _zd, 0)

    @pl.when(sid == 0)
    def _():
        pltpu.sync_copy(degv, deg_sh)

    ones = jnp.ones((LANES,), jnp.float32)

    def _hist(c, _):
        for j in range(CHUNK // LANES):
            cvec = colv[c, pl.ds(j * LANES, LANES)]
            plsc.addupdate_scatter(
                degv, [lax.shift_right_logical(cvec, 4), cvec & 15], ones
            )
        return 0

    lax.fori_loop(0, cpt, _hist, 0)
    plsc.subcore_barrier()

    nred = degv.shape[0] // CHUNK
    for k in range(nred):
        pltpu.sync_copy(
            degv.at[pl.ds(k * CHUNK, CHUNK)], deg_sh.at[idr.at[k]], add=True
        )
    plsc.subcore_barrier()

    nrow16 = rpt // LANES
    pltpu.sync_copy(deg_sh.at[pl.ds(sid * nrow16, nrow16)], degv.at[pl.ds(0, nrow16)])

    def _newton(t, _):
        d = degv[t] + 1.0
        i = plsc.bitcast(d, jnp.int32)
        i = _splat(0x5F3759DF, jnp.int32) - lax.shift_right_logical(i, 1)
        y = plsc.bitcast(i, jnp.float32)
        for _ in range(3):
            y = y * (1.5 - 0.5 * d * y * y)
        disv[pl.ds(t * LANES, LANES)] = y
        dinvv[pl.ds(t * LANES, LANES)] = y * y
        return 0

    lax.fori_loop(0, nrow16, _newton, 0)

    def _scale_rows(kk, src_hbm, dst_hbm, svec, other_hbm, combine):
        sl = pl.ds(base + kk * CHUNK, CHUNK)
        if combine:
            pltpu.sync_copy(acc_sh.at[sl], gbuf.at[0])
            pltpu.sync_copy(other_hbm.at[cid, sl], gbuf.at[1])
        else:
            pltpu.sync_copy(src_hbm.at[cid, sl], gbuf.at[0])

        def _row16(t, _):
            dvec = svec[pl.ds(kk * CHUNK + t * LANES, LANES)]
            for l in range(LANES):
                s = _splat(dvec[l])
                r = t * LANES + l
                for j in range(HD // LANES):
                    cs = pl.ds(j * LANES, LANES)
                    v = gbuf[0, r, cs]
                    if combine:
                        v = v + gbuf[1, r, cs]
                    gbuf[2, r, cs] = v * s
            return 0

        lax.fori_loop(0, CHUNK // LANES, _row16, 0)
        pltpu.sync_copy(gbuf.at[2], dst_hbm.at[cid, sl])

    def _p1(kk, _):
        _scale_rows(kk, x_hbm, g0_hbm, disv, None, False)
        return 0

    lax.fori_loop(0, nck, _p1, 0)
    plsc.subcore_barrier()

    def _hop(src_hbm):
        def gather(cc, bb):
            pltpu.async_copy(src_hbm.at[cid].at[rowv.at[cc]], gbuf.at[bb], sems[bb])

        def gwait(cc, bb):
            pltpu.make_async_copy(
                src_hbm.at[cid].at[rowv.at[cc]], gbuf.at[bb], sems[bb]
            ).wait()

        gather(0, 0)
        gather(1, 1)

        def body(k, _):
            for b in range(NBUF):
                c = k * NBUF + b
                gwait(c, b)

                @pl.when(c + 2 < cpt)
                def _():
                    gather(c + 2, (b + 2) % NBUF)

                pltpu.sync_copy(gbuf.at[b], acc_sh.at[colv.at[c]], add=True)
            return 0

        lax.fori_loop(0, cpt // NBUF, body, 0)
        plsc.subcore_barrier()

    _hop(g0_hbm)

    _zero_buf(3)

    def _p3(kk, _):
        _scale_rows(kk, None, g1_hbm, dinvv, g0_hbm, True)
        pltpu.sync_copy(gbuf.at[3], acc_sh.at[pl.ds(base + kk * CHUNK, CHUNK)])
        return 0

    lax.fori_loop(0, nck, _p3, 0)
    plsc.subcore_barrier()

    _hop(g1_hbm)

    def _p5(kk, _):
        _scale_rows(kk, None, h2_hbm, disv, g1_hbm, True)
        return 0

    lax.fori_loop(0, nck, _p5, 0)


def _make_sc_kernel(npad, cpt):
    shp = jax.ShapeDtypeStruct((NC, npad, HD), jnp.float32)
    return pl.kernel(
        _sc_body,
        out_type=(shp, shp, shp),
        mesh=_sc_mesh(),
        compiler_params=pltpu.CompilerParams(
            needs_layout_passes=False, use_tc_tiling_on_sc=False
        ),
        scratch_types=[
            pltpu.VMEM((cpt, CHUNK), jnp.int32),
            pltpu.VMEM((cpt, CHUNK), jnp.int32),
            pltpu.VMEM((NBUF, CHUNK, HD), jnp.float32),
            pltpu.VMEM((npad // LANES, LANES), jnp.float32),
            pltpu.VMEM((npad // NS,), jnp.float32),
            pltpu.VMEM((npad // NS,), jnp.float32),
            pltpu.VMEM((npad // LANES // CHUNK, CHUNK), jnp.int32),
            pltpu.VMEM_SHARED((npad, HD), jnp.float32),
            pltpu.VMEM_SHARED((npad // LANES, LANES), jnp.float32),
            pltpu.SemaphoreType.DMA,
            pltpu.SemaphoreType.DMA,
            pltpu.SemaphoreType.DMA,
            pltpu.SemaphoreType.DMA,
        ],
    )


def _mm_body(h2_ref, wt_ref, b_ref, out_ref):
    s2 = jnp.concatenate([h2_ref[0], h2_ref[1]], axis=-1)
    out_ref[...] = (
        jnp.dot(s2, wt_ref[...], preferred_element_type=jnp.float32) + b_ref[...]
    )


def _mm_call(h2, wt, bp, npad):
    blk = 1024
    return pl.pallas_call(
        _mm_body,
        grid=(npad // blk,),
        in_specs=[
            pl.BlockSpec((NC, blk, HD), lambda i: (0, i, 0)),
            pl.BlockSpec((128, 128), lambda i: (0, 0)),
            pl.BlockSpec((1, 128), lambda i: (0, 0)),
        ],
        out_specs=pl.BlockSpec((blk, 128), lambda i: (i, 0)),
        out_shape=jax.ShapeDtypeStruct((npad, 128), jnp.float32),
        compiler_params=pltpu.CompilerParams(
            dimension_semantics=("arbitrary",)
        ),
    )(h2, wt, bp)


def kernel(x, edge_index, W, b):
    n, d_feat = x.shape
    n_cls = W.shape[0]
    e = edge_index.shape[1]

    npad = ((n + 512 - 1) // 512) * 512
    unit = NS * CHUNK * NBUF
    epad = ((e + unit - 1) // unit) * unit
    cpt = epad // (NS * CHUNK)
    n_dummy = npad - n

    row = edge_index[0].astype(jnp.int32)
    col = edge_index[1].astype(jnp.int32)
    pad_idx = n + (jnp.arange(epad - e, dtype=jnp.int32) % n_dummy)
    row3 = jnp.concatenate([row, pad_idx]).reshape(NS, cpt, CHUNK)
    col3 = jnp.concatenate([col, pad_idx]).reshape(NS, cpt, CHUNK)

    xs = jnp.zeros((NC, npad, HD), jnp.float32)
    xs = xs.at[0, :n].set(x[:, :HD]).at[1, :n].set(x[:, HD:])
    wt = jnp.zeros((d_feat, 128), jnp.float32).at[:, :n_cls].set(W.T)
    bp = jnp.zeros((1, 128), jnp.float32).at[0, :n_cls].set(b)

    h2, _, _ = _make_sc_kernel(npad, cpt)(xs, row3, col3)
    out_full = _mm_call(h2, wt, bp, npad)
    return out_full[:n, :n_cls]


# speedup vs baseline: 35.2046x; 1.1003x over previous
import jax
import jax.numpy as jnp
from jax import lax
from jax.experimental import pallas as pl
from jax.experimental.pallas import tpu as pltpu
from jax.experimental.pallas import tpu_sc as plsc

NC = 2
NS = 16
LANES = 16
CHUNK = 128
NBUF = 4
HD = 64


def _sc_mesh():
    return plsc.VectorSubcoreMesh(
        core_axis_name="c", subcore_axis_name="s", num_cores=NC, num_subcores=NS
    )


def _splat(s, dtype=jnp.float32):
    return jnp.full((LANES,), s, dtype)


def _sc_body(x_hbm, row_hbm, col_hbm, h2_hbm, g0_hbm, g1_hbm,
             rowv, colv, gbuf, degv, disv, dinvv, idr,
             acc_sh, deg_sh, sem0, sem1, sem2, sem3):
    sems = (sem0, sem1, sem2, sem3)
    cid = lax.axis_index("c")
    sid = lax.axis_index("s")
    cpt = rowv.shape[0]
    npad = acc_sh.shape[0]
    rpt = npad // NS
    nck = rpt // CHUNK
    base = sid * rpt

    pltpu.sync_copy(row_hbm.at[sid], rowv)
    pltpu.sync_copy(col_hbm.at[sid], colv)

    zero16 = jnp.zeros((LANES,), jnp.float32)

    def _zero_buf(bb):
        def _zb(i, _):
            for j in range(HD // LANES):
                gbuf[bb, i, pl.ds(j * LANES, LANES)] = zero16
            return 0

        lax.fori_loop(0, CHUNK, _zb, 0)

    _zero_buf(0)
    for kk in range(nck):
        pltpu.sync_copy(gbuf.at[0], acc_sh.at[pl.ds(base + kk * CHUNK, CHUNK)])

    def _idr(k, _):
        def _idrj(j, _):
            idr[k, pl.ds(j * LANES, LANES)] = (
                jnp.arange(LANES, dtype=jnp.int32)
                + _splat(k * CHUNK + j * LANES, jnp.int32)
            )
            return 0

        lax.fori_loop(0, CHUNK // LANES, _idrj, 0)
        return 0

    lax.fori_loop(0, idr.shape[0], _idr, 0)

    def _zd(i, _):
        degv[i] = zero16
        return 0

    lax.fori_loop(0, degv.shape[0], _zd, 0)

    @pl.when(sid == 0)
    def _():
        pltpu.sync_copy(degv, deg_sh)

    ones = jnp.ones((LANES,), jnp.float32)

    def _hist(c, _):
        for j in range(CHUNK // LANES):
            cvec = colv[c, pl.ds(j * LANES, LANES)]
            plsc.addupdate_scatter(
                degv, [lax.shift_right_logical(cvec, 4), cvec & 15], ones
            )
        return 0

    lax.fori_loop(0, cpt, _hist, 0)
    plsc.subcore_barrier()

    nred = degv.shape[0] // CHUNK
    for k in range(nred):
        pltpu.sync_copy(
            degv.at[pl.ds(k * CHUNK, CHUNK)], deg_sh.at[idr.at[k]], add=True
        )
    plsc.subcore_barrier()

    nrow16 = rpt // LANES
    pltpu.sync_copy(deg_sh.at[pl.ds(sid * nrow16, nrow16)], degv.at[pl.ds(0, nrow16)])

    def _newton(t, _):
        d = degv[t] + 1.0
        i = plsc.bitcast(d, jnp.int32)
        i = _splat(0x5F3759DF, jnp.int32) - lax.shift_right_logical(i, 1)
        y = plsc.bitcast(i, jnp.float32)
        for _ in range(3):
            y = y * (1.5 - 0.5 * d * y * y)
        disv[pl.ds(t * LANES, LANES)] = y
        dinvv[pl.ds(t * LANES, LANES)] = y * y
        return 0

    lax.fori_loop(0, nrow16, _newton, 0)

    half = pl.ds(cid * HD, HD)

    def _scale_rows(kk, src_hbm, dst_hbm, svec, other_hbm, combine, wide_dst):
        sl = pl.ds(base + kk * CHUNK, CHUNK)
        if combine:
            pltpu.sync_copy(acc_sh.at[sl], gbuf.at[0])
            pltpu.sync_copy(other_hbm.at[cid, sl], gbuf.at[1])
        else:
            pltpu.sync_copy(src_hbm.at[sl, half], gbuf.at[0])

        def _row16(t, _):
            dvec = svec[pl.ds(kk * CHUNK + t * LANES, LANES)]
            for l in range(LANES):
                s = _splat(dvec[l])
                r = t * LANES + l
                for j in range(HD // LANES):
                    cs = pl.ds(j * LANES, LANES)
                    v = gbuf[0, r, cs]
                    if combine:
                        v = v + gbuf[1, r, cs]
                    gbuf[2, r, cs] = v * s
            return 0

        lax.fori_loop(0, CHUNK // LANES, _row16, 0)
        if wide_dst:
            pltpu.sync_copy(gbuf.at[2], dst_hbm.at[sl, half])
        else:
            pltpu.sync_copy(gbuf.at[2], dst_hbm.at[cid, sl])

    def _p1(kk, _):
        _scale_rows(kk, x_hbm, g0_hbm, disv, None, False, False)
        return 0

    lax.fori_loop(0, nck, _p1, 0)
    plsc.subcore_barrier()

    def _hop(src_hbm):
        def gather(cc, bb):
            pltpu.async_copy(src_hbm.at[cid].at[rowv.at[cc]], gbuf.at[bb], sems[bb])

        def gwait(cc, bb):
            pltpu.make_async_copy(
                src_hbm.at[cid].at[rowv.at[cc]], gbuf.at[bb], sems[bb]
            ).wait()

        gather(0, 0)
        gather(1, 1)

        def body(k, _):
            for b in range(NBUF):
                c = k * NBUF + b
                gwait(c, b)

                @pl.when(c + 2 < cpt)
                def _():
                    gather(c + 2, (b + 2) % NBUF)

                pltpu.sync_copy(gbuf.at[b], acc_sh.at[colv.at[c]], add=True)
            return 0

        lax.fori_loop(0, cpt // NBUF, body, 0)
        plsc.subcore_barrier()

    _hop(g0_hbm)

    _zero_buf(3)

    def _p3(kk, _):
        _scale_rows(kk, None, g1_hbm, dinvv, g0_hbm, True, False)
        pltpu.sync_copy(gbuf.at[3], acc_sh.at[pl.ds(base + kk * CHUNK, CHUNK)])
        return 0

    lax.fori_loop(0, nck, _p3, 0)
    plsc.subcore_barrier()

    _hop(g1_hbm)

    def _p5(kk, _):
        _scale_rows(kk, None, h2_hbm, disv, g1_hbm, True, True)
        return 0

    lax.fori_loop(0, nck, _p5, 0)


def _make_sc_kernel(npad, cpt):
    shp = jax.ShapeDtypeStruct((NC, npad, HD), jnp.float32)
    return pl.kernel(
        _sc_body,
        out_type=(
            jax.ShapeDtypeStruct((npad, 2 * HD), jnp.float32),
            shp,
            shp,
        ),
        mesh=_sc_mesh(),
        compiler_params=pltpu.CompilerParams(
            needs_layout_passes=False, use_tc_tiling_on_sc=False
        ),
        scratch_types=[
            pltpu.VMEM((cpt, CHUNK), jnp.int32),
            pltpu.VMEM((cpt, CHUNK), jnp.int32),
            pltpu.VMEM((NBUF, CHUNK, HD), jnp.float32),
            pltpu.VMEM((npad // LANES, LANES), jnp.float32),
            pltpu.VMEM((npad // NS,), jnp.float32),
            pltpu.VMEM((npad // NS,), jnp.float32),
            pltpu.VMEM((npad // LANES // CHUNK, CHUNK), jnp.int32),
            pltpu.VMEM_SHARED((npad, HD), jnp.float32),
            pltpu.VMEM_SHARED((npad // LANES, LANES), jnp.float32),
            pltpu.SemaphoreType.DMA,
            pltpu.SemaphoreType.DMA,
            pltpu.SemaphoreType.DMA,
            pltpu.SemaphoreType.DMA,
        ],
    )


def _mm_body(h2_ref, wt_ref, b_ref, out_ref):
    out_ref[...] = (
        jnp.dot(h2_ref[...], wt_ref[...], preferred_element_type=jnp.float32)
        + b_ref[...]
    )


def _mm_call(h2, wt, bp, n, n_cls, d_feat):
    blk = 1024
    grid = (n + blk - 1) // blk
    return pl.pallas_call(
        _mm_body,
        grid=(grid,),
        in_specs=[
            pl.BlockSpec((blk, d_feat), lambda i: (i, 0)),
            pl.BlockSpec((d_feat, n_cls), lambda i: (0, 0)),
            pl.BlockSpec((1, n_cls), lambda i: (0, 0)),
        ],
        out_specs=pl.BlockSpec((blk, n_cls), lambda i: (i, 0)),
        out_shape=jax.ShapeDtypeStruct((n, n_cls), jnp.float32),
        compiler_params=pltpu.CompilerParams(
            dimension_semantics=("arbitrary",)
        ),
    )(h2, wt, bp)


def kernel(x, edge_index, W, b):
    n, d_feat = x.shape
    n_cls = W.shape[0]
    e = edge_index.shape[1]

    npad = ((n + 512 - 1) // 512) * 512
    unit = NS * CHUNK * NBUF
    epad = ((e + unit - 1) // unit) * unit
    cpt = epad // (NS * CHUNK)
    n_dummy = npad - n

    row = edge_index[0].astype(jnp.int32)
    col = edge_index[1].astype(jnp.int32)
    pad_idx = n + (jnp.arange(epad - e, dtype=jnp.int32) % n_dummy)
    row3 = jnp.concatenate([row, pad_idx]).reshape(NS, cpt, CHUNK)
    col3 = jnp.concatenate([col, pad_idx]).reshape(NS, cpt, CHUNK)

    xs = jnp.zeros((npad, d_feat), jnp.float32).at[:n].set(x)
    wt = W.T
    bp = b.reshape(1, n_cls)

    h2, _, _ = _make_sc_kernel(npad, cpt)(xs, row3, col3)
    return _mm_call(h2, wt, bp, n, n_cls, d_feat)
